# Initial kernel scaffold; baseline (speedup 1.0000x reference)
import jax
import jax.numpy as jnp
from jax.experimental import pallas as pl


def kernel(x, token_table, pos_table):
    raise NotImplementedError("write your pallas kernel here")


# speedup vs baseline: 6.3221x; 6.3221x over previous
import functools

import jax
import jax.numpy as jnp
from jax import lax
from jax.experimental import pallas as pl
from jax.experimental.pallas import tpu as pltpu
from jax.experimental.pallas import tpu_sc as plsc

BATCH = 4096
SEQ = 200
EMBED = 64
LANES = 16

NUM_CORES = 2
NUM_SUBCORES = 16
NUM_WORKERS = NUM_CORES * NUM_SUBCORES
ROWS_PER_WORKER = BATCH // NUM_WORKERS

SEQ_A = 128
SEQ_B = SEQ - SEQ_A


def _body(x_hbm, tok_hbm, pos_hbm, out_hbm, idx_a, idx_b, pos_v, rows_v, sem):
    wid = lax.axis_index("s") * NUM_CORES + lax.axis_index("c")
    base = wid * ROWS_PER_WORKER

    pltpu.sync_copy(x_hbm.at[pl.ds(base, ROWS_PER_WORKER), pl.ds(0, SEQ_A)], idx_a)
    pltpu.sync_copy(x_hbm.at[pl.ds(base, ROWS_PER_WORKER), pl.ds(SEQ_A, SEQ_B)], idx_b)
    pltpu.sync_copy(pos_hbm, pos_v)

    @pl.loop(0, ROWS_PER_WORKER)
    def _row(r):
        cp_a = pltpu.async_copy(
            tok_hbm.at[idx_a.at[r]], rows_v.at[pl.ds(0, SEQ_A), :], sem
        )
        cp_b = pltpu.async_copy(
            tok_hbm.at[idx_b.at[r]], rows_v.at[pl.ds(SEQ_A, SEQ_B), :], sem
        )
        cp_a.wait()
        cp_b.wait()

        @pl.loop(0, SEQ, unroll=8)
        def _add(s):
            for c in range(EMBED // LANES):
                col = pl.ds(c * LANES, LANES)
                plsc.addupdate(rows_v.at[s, col], pos_v[s, col])

        pltpu.sync_copy(rows_v, out_hbm.at[base + r])


def kernel(x, token_table, pos_table):
    x = x.astype(jnp.int32)
    mesh = plsc.VectorSubcoreMesh(
        core_axis_name="c", subcore_axis_name="s",
        num_cores=NUM_CORES, num_subcores=NUM_SUBCORES,
    )
    run = pl.kernel(
        _body,
        out_type=jax.ShapeDtypeStruct((BATCH, SEQ, EMBED), jnp.float32),
        mesh=mesh,
        compiler_params=pltpu.CompilerParams(use_tc_tiling_on_sc=False),
        scratch_types=[
            pltpu.VMEM((ROWS_PER_WORKER, SEQ_A), jnp.int32),
            pltpu.VMEM((ROWS_PER_WORKER, SEQ_B), jnp.int32),
            pltpu.VMEM((SEQ, EMBED), jnp.float32),
            pltpu.VMEM((SEQ, EMBED), jnp.float32),
            pltpu.SemaphoreType.DMA,
        ],
    )
    return run(x, token_table, pos_table)


# speedup vs baseline: 7.6749x; 1.2140x over previous
import jax
import jax.numpy as jnp
from jax import lax
from jax.experimental import pallas as pl
from jax.experimental.pallas import tpu as pltpu
from jax.experimental.pallas import tpu_sc as plsc

BATCH = 4096
SEQ = 200
EMBED = 64
LANES = 16

NUM_CORES = 2
NUM_SUBCORES = 16
NUM_WORKERS = NUM_CORES * NUM_SUBCORES
ROWS_PER_WORKER = BATCH // NUM_WORKERS

SEQ_A = 128
SEQ_B = SEQ - SEQ_A
NB = 4


def _body(x_hbm, tok_hbm, pos_hbm, out_hbm, idx_a, idx_b, pos_v, rows_v,
          gsem, ssem):
    wid = lax.axis_index("s") * NUM_CORES + lax.axis_index("c")
    base = wid * ROWS_PER_WORKER

    pltpu.sync_copy(x_hbm.at[pl.ds(base, ROWS_PER_WORKER), pl.ds(0, SEQ_A)], idx_a)
    pltpu.sync_copy(x_hbm.at[pl.ds(base, ROWS_PER_WORKER), pl.ds(SEQ_A, SEQ_B)], idx_b)
    pltpu.sync_copy(pos_hbm, pos_v)

    def start_gather(r, b):
        pltpu.async_copy(tok_hbm.at[idx_a.at[r]],
                         rows_v.at[b, pl.ds(0, SEQ_A), :], gsem.at[b])
        pltpu.async_copy(tok_hbm.at[idx_b.at[r]],
                         rows_v.at[b, pl.ds(SEQ_A, SEQ_B), :], gsem.at[b])

    def wait_gather(r, b):
        pltpu.make_async_copy(tok_hbm.at[idx_a.at[r]],
                              rows_v.at[b, pl.ds(0, SEQ_A), :], gsem.at[b]).wait()
        pltpu.make_async_copy(tok_hbm.at[idx_b.at[r]],
                              rows_v.at[b, pl.ds(SEQ_A, SEQ_B), :], gsem.at[b]).wait()

    def drain_store(b):
        pltpu.make_async_copy(rows_v.at[b], out_hbm.at[0], ssem.at[b]).wait()

    for b in range(NB - 1):
        start_gather(b, b)

    @pl.loop(0, ROWS_PER_WORKER, step=NB)
    def _rows(k):
        for b in range(NB):
            r = k + b
            q = r + NB - 1
            nb = (b + NB - 1) % NB

            @pl.when(q < ROWS_PER_WORKER)
            def _prefetch():
                @pl.when(q >= NB)
                def _drain():
                    drain_store(nb)

                start_gather(q, nb)

            wait_gather(r, b)

            @pl.loop(0, SEQ, unroll=8)
            def _add(s):
                for c in range(EMBED // LANES):
                    col = pl.ds(c * LANES, LANES)
                    plsc.addupdate(rows_v.at[b, s, col], pos_v[s, col])

            pltpu.async_copy(rows_v.at[b], out_hbm.at[base + r], ssem.at[b])

    for b in range(NB):
        drain_store(b)


def kernel(x, token_table, pos_table):
    x = x.astype(jnp.int32)
    mesh = plsc.VectorSubcoreMesh(
        core_axis_name="c", subcore_axis_name="s",
        num_cores=NUM_CORES, num_subcores=NUM_SUBCORES,
    )
    run = pl.kernel(
        _body,
        out_type=jax.ShapeDtypeStruct((BATCH, SEQ, EMBED), jnp.float32),
        mesh=mesh,
        compiler_params=pltpu.CompilerParams(use_tc_tiling_on_sc=False),
        scratch_types=[
            pltpu.VMEM((ROWS_PER_WORKER, SEQ_A), jnp.int32),
            pltpu.VMEM((ROWS_PER_WORKER, SEQ_B), jnp.int32),
            pltpu.VMEM((SEQ, EMBED), jnp.float32),
            pltpu.VMEM((NB, SEQ, EMBED), jnp.float32),
            pltpu.SemaphoreType.DMA((NB,)),
            pltpu.SemaphoreType.DMA((NB,)),
        ],
    )
    return run(x, token_table, pos_table)
